# baseline (speedup 1.0000x reference)
import functools

import jax
import jax.numpy as jnp
from jax import lax
from jax.experimental import pallas as pl
from jax.experimental.pallas import tpu as pltpu
from jax.experimental.pallas import tpu_sc as plsc

_EMB = 32
_B = 16384
_NF = 19
_NC = 2
_NS = 16
_NW = _NC * _NS
_BPW = _B // _NW
_NBUF = 6


def _make_body(nf):
    def _body(xt_hbm, *refs):
        tables = refs[:nf]
        out_hbm = refs[nf]
        rest = refs[nf + 1:]
        idx_v = rest[:nf]
        rows_v = rest[nf:nf + _NBUF]
        isem, gsem, wsem = rest[nf + _NBUF:]
        wid = lax.axis_index("s") * _NC + lax.axis_index("c")
        base = wid * _BPW

        idescr = [
            pltpu.async_copy(xt_hbm.at[pl.ds(i * _B + base, _BPW)],
                             idx_v[i], isem)
            for i in range(nf)
        ]
        for d in idescr:
            d.wait()

        def gather(i):
            return pltpu.async_copy(tables[i].at[idx_v[i]],
                                    rows_v[i % _NBUF], gsem)

        def write(i):
            return pltpu.async_copy(
                rows_v[i % _NBUF],
                out_hbm.at[pl.ds(base, _BPW), pl.ds(i * _EMB, _EMB)], wsem)

        gd = [gather(i) for i in range(min(_NBUF, nf))]
        wd = []
        for i in range(nf):
            gd[i].wait()
            wd.append(write(i))
            j = i + _NBUF
            if j < nf:
                wd[i].wait()
                gd.append(gather(j))
        for i in range(max(0, nf - _NBUF), nf):
            wd[i].wait()

    return _body


def _make_lookup(nf):
    return functools.partial(
        pl.kernel,
        out_type=jax.ShapeDtypeStruct((_B, nf * _EMB), jnp.float32),
        mesh=plsc.VectorSubcoreMesh(core_axis_name="c", subcore_axis_name="s"),
        compiler_params=pltpu.CompilerParams(use_tc_tiling_on_sc=False),
        scratch_types=(
            [pltpu.VMEM((_BPW,), jnp.int32) for _ in range(nf)]
            + [pltpu.VMEM((_BPW, _EMB), jnp.float32) for _ in range(_NBUF)]
            + [pltpu.SemaphoreType.DMA] * 3
        ),
    )(_make_body(nf))


_NF_A = 10
_NF_B = _NF - _NF_A
_lookup_a = _make_lookup(_NF_A)
_lookup_b = _make_lookup(_NF_B)


def kernel(x, W0, W1, W2, W3, W4, W5, W6, W7, W8, W9, W10, W11, W12, W13,
           W14, W15, W16, W17, W18):
    tables = (W0, W1, W2, W3, W4, W5, W6, W7, W8, W9, W10, W11, W12, W13,
              W14, W15, W16, W17, W18)
    xt = x.T
    xt_a = xt[:_NF_A].reshape(-1)
    xt_b = xt[_NF_A:].reshape(-1)
    out_a = _lookup_a(xt_a, *tables[:_NF_A])
    out_b = _lookup_b(xt_b, *tables[_NF_A:])
    out = jnp.concatenate(
        [out_a.reshape(_B, _NF_A, _EMB), out_b.reshape(_B, _NF_B, _EMB)],
        axis=1)
    return out


# speedup vs baseline: 1.0166x; 1.0166x over previous
import functools

import jax
import jax.numpy as jnp
from jax import lax
from jax.experimental import pallas as pl
from jax.experimental.pallas import tpu as pltpu
from jax.experimental.pallas import tpu_sc as plsc

_EMB = 32
_B = 16384
_NF = 19
_NC = 2
_NS = 16
_NW = _NC * _NS
_BPW = _B // _NW
_NBUF = 6


def _body(xt_hbm, *refs):
    tables = refs[:_NF]
    out_hbm = refs[_NF]
    rest = refs[_NF + 1:]
    idx_v = rest[:_NF]
    rows_v = rest[_NF:_NF + _NBUF]
    isem, gsem, wsem = rest[_NF + _NBUF:]
    wid = lax.axis_index("s") * _NC + lax.axis_index("c")
    base = wid * _BPW

    idescr = [
        pltpu.async_copy(xt_hbm.at[pl.ds(i * _B + base, _BPW)], idx_v[i], isem)
        for i in range(_NF)
    ]
    for d in idescr:
        d.wait()

    def gather(i):
        return pltpu.async_copy(tables[i].at[idx_v[i]],
                                rows_v[i % _NBUF], gsem)

    def write(i):
        return pltpu.async_copy(
            rows_v[i % _NBUF],
            out_hbm.at[pl.ds(base, _BPW), pl.ds(i * _EMB, _EMB)], wsem)

    gd = [gather(i) for i in range(_NBUF)]
    wd = []
    for i in range(_NF):
        gd[i].wait()
        wd.append(write(i))
        j = i + _NBUF
        if j < _NF:
            wd[i].wait()
            gd.append(gather(j))
    for i in range(_NF - _NBUF, _NF):
        wd[i].wait()


_sc_lookup = functools.partial(
    pl.kernel,
    out_type=jax.ShapeDtypeStruct((_B, _NF * _EMB), jnp.float32),
    mesh=plsc.VectorSubcoreMesh(core_axis_name="c", subcore_axis_name="s"),
    compiler_params=pltpu.CompilerParams(use_tc_tiling_on_sc=False),
    scratch_types=(
        [pltpu.VMEM((_BPW,), jnp.int32) for _ in range(_NF)]
        + [pltpu.VMEM((_BPW, _EMB), jnp.float32) for _ in range(_NBUF)]
        + [pltpu.SemaphoreType.DMA] * 3
    ),
)(_body)


def kernel(x, W0, W1, W2, W3, W4, W5, W6, W7, W8, W9, W10, W11, W12, W13,
           W14, W15, W16, W17, W18):
    xt = x.T.reshape(-1)
    out = _sc_lookup(xt, W0, W1, W2, W3, W4, W5, W6, W7, W8, W9, W10, W11,
                     W12, W13, W14, W15, W16, W17, W18)
    return out.reshape(_B, _NF, _EMB)
